# baseline (speedup 1.0000x reference)
import jax
import jax.numpy as jnp
from jax import lax
from jax.experimental import pallas as pl
from jax.experimental.pallas import tpu as pltpu
from jax.experimental.pallas import tpu_sc as plsc

_NUM_BINS = 64
_NCAT = _NUM_BINS + 1
_NC = 2
_NS = 16
_NW = _NC * _NS
_L = 16
_CHUNK = 512
_GROUPS = _CHUNK // _L
_CWORDS = _CHUNK * _NCAT


def _wid():
    return lax.axis_index("s") * _NC + lax.axis_index("c")


def _minmax_body(x_hbm, part_hbm, xbuf, pbuf):
    w = _wid()
    rows = x_hbm.shape[0] // _NW
    pltpu.sync_copy(x_hbm.at[pl.ds(w * rows, rows)], xbuf)

    def step(i, carry):
        vmin, vmax = carry
        xv = xbuf[pl.ds(i * _L, _L)]
        return jnp.minimum(vmin, xv), jnp.maximum(vmax, xv)

    init = (xbuf[pl.ds(0, _L)], xbuf[pl.ds(0, _L)])
    vmin, vmax = lax.fori_loop(1, rows // _L, step, init, unroll=8)
    pbuf[0, :] = vmin
    pbuf[1, :] = vmax
    pltpu.sync_copy(pbuf, part_hbm.at[w])


def _expand_body(x_hbm, part_hbm, zeros_hbm, out_hbm,
                 pv, xbuf, buf0, buf1, idx0, idx1, sem0, sem1):
    w = _wid()
    rows = x_hbm.shape[0] // _NW
    nchunks = rows // _CHUNK
    base = w * rows

    pltpu.sync_copy(part_hbm, pv)
    vmin = pv[0, 0, :]
    vmax = pv[0, 1, :]
    for k in range(1, _NW):
        vmin = jnp.minimum(vmin, pv[k, 0, :])
        vmax = jnp.maximum(vmax, pv[k, 1, :])
    iota = lax.iota(jnp.int32, _L)
    for k in (1, 2, 4, 8):
        perm = jnp.bitwise_xor(iota, k)
        vmin = jnp.minimum(vmin, vmin.at[perm].get(mode="promise_in_bounds"))
        vmax = jnp.maximum(vmax, vmax.at[perm].get(mode="promise_in_bounds"))
    lo = vmin
    delta = (vmax - vmin) / jnp.float32(_NUM_BINS - 1)
    inv = jnp.float32(1.0) / delta

    pltpu.sync_copy(zeros_hbm, buf0)
    pltpu.sync_copy(zeros_hbm, buf1)

    onev = jnp.full((_L,), 1.0, dtype=jnp.float32)
    zerov = jnp.zeros((_L,), dtype=jnp.float32)
    laneoff = iota * _NCAT

    def process(c, bufr, idxr, sem):
        @pl.when(c >= 2)
        def _():
            pltpu.make_async_copy(
                bufr, out_hbm.at[pl.ds(0, _CWORDS)], sem).wait()
            for r in range(_GROUPS):
                idxv = idxr[pl.ds(r * _L, _L)]
                flat = idxv + laneoff + (r * _L * _NCAT)
                plsc.store_scatter(bufr, [flat], zerov)

        pltpu.sync_copy(x_hbm.at[pl.ds(base + c * _CHUNK, _CHUNK)], xbuf)
        for r in range(_GROUPS):
            xv = xbuf[pl.ds(r * _L, _L)]
            t = (xv - lo) * inv
            idxv = jnp.minimum(t.astype(jnp.int32) + 1, _NUM_BINS)
            idxr[pl.ds(r * _L, _L)] = idxv
            flat = idxv + laneoff + (r * _L * _NCAT)
            plsc.store_scatter(bufr, [flat], onev)
        pltpu.make_async_copy(
            bufr,
            out_hbm.at[pl.ds((base + c * _CHUNK) * _NCAT, _CWORDS)],
            sem,
        ).start()

    def body(c, _):
        slot = lax.rem(c, 2)

        @pl.when(slot == 0)
        def _():
            process(c, buf0, idx0, sem0)

        @pl.when(slot == 1)
        def _():
            process(c, buf1, idx1, sem1)

        return 0

    lax.fori_loop(0, nchunks, body, 0)
    pltpu.make_async_copy(buf0, out_hbm.at[pl.ds(0, _CWORDS)], sem0).wait()
    pltpu.make_async_copy(buf1, out_hbm.at[pl.ds(0, _CWORDS)], sem1).wait()


def kernel(feature):
    n = feature.shape[0]
    mesh = plsc.VectorSubcoreMesh(core_axis_name="c", subcore_axis_name="s")

    cparams = pltpu.CompilerParams(needs_layout_passes=False)
    minmax = pl.kernel(
        _minmax_body,
        mesh=mesh,
        compiler_params=cparams,
        out_type=jax.ShapeDtypeStruct((_NW, 2, _L), jnp.float32),
        scratch_types=[
            pltpu.VMEM((n // _NW,), jnp.float32),
            pltpu.VMEM((2, _L), jnp.float32),
        ],
    )
    part = minmax(feature)

    expand = pl.kernel(
        _expand_body,
        mesh=mesh,
        compiler_params=cparams,
        out_type=jax.ShapeDtypeStruct((n * _NCAT,), jnp.float32),
        scratch_types=[
            pltpu.VMEM((_NW, 2, _L), jnp.float32),
            pltpu.VMEM((_CHUNK,), jnp.float32),
            pltpu.VMEM((_CWORDS,), jnp.float32),
            pltpu.VMEM((_CWORDS,), jnp.float32),
            pltpu.VMEM((_CHUNK,), jnp.int32),
            pltpu.VMEM((_CHUNK,), jnp.int32),
            pltpu.SemaphoreType.DMA,
            pltpu.SemaphoreType.DMA,
        ],
    )
    zeros = jnp.zeros((_CWORDS,), jnp.float32)
    flat = expand(feature, part, zeros)
    return flat.reshape(n, _NCAT)


# speedup vs baseline: 1.1814x; 1.1814x over previous
import jax
import jax.numpy as jnp
from jax import lax
from jax.experimental import pallas as pl
from jax.experimental.pallas import tpu as pltpu
from jax.experimental.pallas import tpu_sc as plsc

_NUM_BINS = 64
_NCAT = _NUM_BINS + 1
_NC = 2
_NS = 16
_NW = _NC * _NS
_L = 16
_CHUNK = 512
_GROUPS = _CHUNK // _L
_CWORDS = _CHUNK * _NCAT


def _wid():
    return lax.axis_index("s") * _NC + lax.axis_index("c")


def _minmax_body(x_hbm, part_hbm, xbuf, pbuf):
    w = _wid()
    rows = x_hbm.shape[0] // _NW
    pltpu.sync_copy(x_hbm.at[pl.ds(w * rows, rows)], xbuf)

    def step(i, carry):
        vmin, vmax = carry
        xv = xbuf[pl.ds(i * _L, _L)]
        return jnp.minimum(vmin, xv), jnp.maximum(vmax, xv)

    init = (xbuf[pl.ds(0, _L)], xbuf[pl.ds(0, _L)])
    vmin, vmax = lax.fori_loop(1, rows // _L, step, init, unroll=8)
    pbuf[0, :] = vmin
    pbuf[1, :] = vmax
    pltpu.sync_copy(pbuf, part_hbm.at[w])


def _expand_body(x_hbm, part_hbm, zeros_hbm, out_hbm,
                 pv, xbuf, buf0, buf1, idx0, idx1, sem0, sem1):
    w = _wid()
    rows = x_hbm.shape[0] // _NW
    nchunks = rows // _CHUNK
    base = w * rows

    pltpu.sync_copy(part_hbm, pv)
    vmin = pv[0, 0, :]
    vmax = pv[0, 1, :]
    for k in range(1, _NW):
        vmin = jnp.minimum(vmin, pv[k, 0, :])
        vmax = jnp.maximum(vmax, pv[k, 1, :])
    iota = lax.iota(jnp.int32, _L)
    for k in (1, 2, 4, 8):
        perm = jnp.bitwise_xor(iota, k)
        vmin = jnp.minimum(vmin, vmin.at[perm].get(mode="promise_in_bounds"))
        vmax = jnp.maximum(vmax, vmax.at[perm].get(mode="promise_in_bounds"))
    lo = vmin
    delta = (vmax - vmin) / jnp.float32(_NUM_BINS - 1)
    inv = jnp.float32(1.0) / delta

    pltpu.sync_copy(zeros_hbm, buf0)
    pltpu.sync_copy(zeros_hbm, buf1)

    onev = jnp.full((_L,), 1.0, dtype=jnp.float32)
    zerov = jnp.zeros((_L,), dtype=jnp.float32)

    def process(c, bufr, idxr, sem):
        @pl.when(c >= 2)
        def _():
            pltpu.make_async_copy(
                bufr, out_hbm.at[pl.ds(0, _CHUNK), :], sem).wait()
            for r in range(_GROUPS):
                rowv = iota + (r * _L)
                idxv = idxr[pl.ds(r * _L, _L)]
                plsc.store_scatter(bufr, [rowv, idxv], zerov)

        pltpu.sync_copy(x_hbm.at[pl.ds(base + c * _CHUNK, _CHUNK)], xbuf)
        for r in range(_GROUPS):
            xv = xbuf[pl.ds(r * _L, _L)]
            t = (xv - lo) * inv
            idxv = jnp.minimum(t.astype(jnp.int32) + 1, _NUM_BINS)
            idxr[pl.ds(r * _L, _L)] = idxv
            rowv = iota + (r * _L)
            plsc.store_scatter(bufr, [rowv, idxv], onev)
        pltpu.make_async_copy(
            bufr,
            out_hbm.at[pl.ds(base + c * _CHUNK, _CHUNK), :],
            sem,
        ).start()

    def body(c, _):
        slot = lax.rem(c, 2)

        @pl.when(slot == 0)
        def _():
            process(c, buf0, idx0, sem0)

        @pl.when(slot == 1)
        def _():
            process(c, buf1, idx1, sem1)

        return 0

    lax.fori_loop(0, nchunks, body, 0)
    pltpu.make_async_copy(buf0, out_hbm.at[pl.ds(0, _CHUNK), :], sem0).wait()
    pltpu.make_async_copy(buf1, out_hbm.at[pl.ds(0, _CHUNK), :], sem1).wait()


def kernel(feature):
    n = feature.shape[0]
    mesh = plsc.VectorSubcoreMesh(core_axis_name="c", subcore_axis_name="s")

    cparams = pltpu.CompilerParams(needs_layout_passes=False,
                                   use_tc_tiling_on_sc=False)
    minmax = pl.kernel(
        _minmax_body,
        mesh=mesh,
        compiler_params=cparams,
        out_type=jax.ShapeDtypeStruct((_NW, 2, _L), jnp.float32),
        scratch_types=[
            pltpu.VMEM((n // _NW,), jnp.float32),
            pltpu.VMEM((2, _L), jnp.float32),
        ],
    )
    part = minmax(feature)

    expand = pl.kernel(
        _expand_body,
        mesh=mesh,
        compiler_params=cparams,
        out_type=jax.ShapeDtypeStruct((n, _NCAT), jnp.float32),
        scratch_types=[
            pltpu.VMEM((_NW, 2, _L), jnp.float32),
            pltpu.VMEM((_CHUNK,), jnp.float32),
            pltpu.VMEM((_CHUNK, _NCAT), jnp.float32),
            pltpu.VMEM((_CHUNK, _NCAT), jnp.float32),
            pltpu.VMEM((_CHUNK,), jnp.int32),
            pltpu.VMEM((_CHUNK,), jnp.int32),
            pltpu.SemaphoreType.DMA,
            pltpu.SemaphoreType.DMA,
        ],
    )
    zeros = jnp.zeros((_CHUNK, _NCAT), jnp.float32)
    return expand(feature, part, zeros)


# speedup vs baseline: 2.0043x; 1.6966x over previous
import jax
import jax.numpy as jnp
from jax import lax
from jax.experimental import pallas as pl
from jax.experimental.pallas import tpu as pltpu
from jax.experimental.pallas import tpu_sc as plsc

_NUM_BINS = 64
_NCAT = _NUM_BINS + 1
_NC = 2
_NS = 16
_NW = _NC * _NS
_L = 16
_CHUNK = 256
_GROUPS = _CHUNK // _L
_CWORDS = _CHUNK * _NCAT


def _wid():
    return lax.axis_index("s") * _NC + lax.axis_index("c")


def _minmax_body(x_hbm, part_hbm, xbuf, pbuf):
    w = _wid()
    rows = x_hbm.shape[0] // _NW
    pltpu.sync_copy(x_hbm.at[pl.ds(w * rows, rows)], xbuf)

    def step(i, carry):
        vmin, vmax = carry
        xv = xbuf[pl.ds(i * _L, _L)]
        return jnp.minimum(vmin, xv), jnp.maximum(vmax, xv)

    init = (xbuf[pl.ds(0, _L)], xbuf[pl.ds(0, _L)])
    vmin, vmax = lax.fori_loop(1, rows // _L, step, init, unroll=8)
    pbuf[0, :] = vmin
    pbuf[1, :] = vmax
    pltpu.sync_copy(pbuf, part_hbm.at[w])


def _expand_body(x_hbm, part_hbm, zeros_hbm, out_hbm,
                 pv, xbuf, buf0, buf1, idx0, idx1, sem0, sem1):
    w = _wid()
    rows = x_hbm.shape[0] // _NW
    nchunks = rows // _CHUNK
    base = w * rows

    pltpu.sync_copy(part_hbm, pv)
    vmin = pv[0, 0, :]
    vmax = pv[0, 1, :]
    for k in range(1, _NW):
        vmin = jnp.minimum(vmin, pv[k, 0, :])
        vmax = jnp.maximum(vmax, pv[k, 1, :])
    iota = lax.iota(jnp.int32, _L)
    for k in (1, 2, 4, 8):
        perm = jnp.bitwise_xor(iota, k)
        vmin = jnp.minimum(vmin, vmin.at[perm].get(mode="promise_in_bounds"))
        vmax = jnp.maximum(vmax, vmax.at[perm].get(mode="promise_in_bounds"))
    lo = vmin
    delta = (vmax - vmin) / jnp.float32(_NUM_BINS - 1)
    inv = jnp.float32(1.0) / delta

    pltpu.sync_copy(zeros_hbm, buf0)
    pltpu.sync_copy(zeros_hbm, buf1)

    onev = jnp.full((_L,), 1.0, dtype=jnp.float32)
    zerov = jnp.zeros((_L,), dtype=jnp.float32)

    def process(c, bufr, idxr, sem):
        @pl.when(c >= 2)
        def _():
            pltpu.make_async_copy(
                bufr, out_hbm.at[pl.ds(0, _CHUNK), :], sem).wait()
            for r in range(_GROUPS):
                rowv = iota + (r * _L)
                idxv = idxr[pl.ds(r * _L, _L)]
                plsc.store_scatter(bufr, [rowv, idxv], zerov)

        pltpu.sync_copy(x_hbm.at[pl.ds(base + c * _CHUNK, _CHUNK)], xbuf)
        for r in range(_GROUPS):
            xv = xbuf[pl.ds(r * _L, _L)]
            t = (xv - lo) * inv
            idxv = jnp.minimum(t.astype(jnp.int32) + 1, _NUM_BINS)
            idxr[pl.ds(r * _L, _L)] = idxv
            rowv = iota + (r * _L)
            plsc.store_scatter(bufr, [rowv, idxv], onev)
        pltpu.make_async_copy(
            bufr,
            out_hbm.at[pl.ds(base + c * _CHUNK, _CHUNK), :],
            sem,
        ).start()

    def body(c, _):
        slot = lax.rem(c, 2)

        @pl.when(slot == 0)
        def _():
            process(c, buf0, idx0, sem0)

        @pl.when(slot == 1)
        def _():
            process(c, buf1, idx1, sem1)

        return 0

    lax.fori_loop(0, nchunks, body, 0)
    pltpu.make_async_copy(buf0, out_hbm.at[pl.ds(0, _CHUNK), :], sem0).wait()
    pltpu.make_async_copy(buf1, out_hbm.at[pl.ds(0, _CHUNK), :], sem1).wait()


def kernel(feature):
    n = feature.shape[0]
    mesh = plsc.VectorSubcoreMesh(core_axis_name="c", subcore_axis_name="s")

    cparams = pltpu.CompilerParams(needs_layout_passes=False,
                                   use_tc_tiling_on_sc=True)
    minmax = pl.kernel(
        _minmax_body,
        mesh=mesh,
        compiler_params=cparams,
        out_type=jax.ShapeDtypeStruct((_NW, 2, _L), jnp.float32),
        scratch_types=[
            pltpu.VMEM((n // _NW,), jnp.float32),
            pltpu.VMEM((2, _L), jnp.float32),
        ],
    )
    part = minmax(feature)

    expand = pl.kernel(
        _expand_body,
        mesh=mesh,
        compiler_params=cparams,
        out_type=jax.ShapeDtypeStruct((n, _NCAT), jnp.float32),
        scratch_types=[
            pltpu.VMEM((_NW, 2, _L), jnp.float32),
            pltpu.VMEM((_CHUNK,), jnp.float32),
            pltpu.VMEM((_CHUNK, _NCAT), jnp.float32),
            pltpu.VMEM((_CHUNK, _NCAT), jnp.float32),
            pltpu.VMEM((_CHUNK,), jnp.int32),
            pltpu.VMEM((_CHUNK,), jnp.int32),
            pltpu.SemaphoreType.DMA,
            pltpu.SemaphoreType.DMA,
        ],
    )
    zeros = jnp.zeros((_CHUNK, _NCAT), jnp.float32)
    return expand(feature, part, zeros)


# speedup vs baseline: 2.0072x; 1.0015x over previous
import jax
import jax.numpy as jnp
from jax import lax
from jax.experimental import pallas as pl
from jax.experimental.pallas import tpu as pltpu
from jax.experimental.pallas import tpu_sc as plsc

_NUM_BINS = 64
_NCAT = _NUM_BINS + 1
_NC = 2
_NS = 16
_NW = _NC * _NS
_L = 16
_CHUNK = 256
_GROUPS = _CHUNK // _L
_MMSUB = 16384


def _body(x_hbm, zeros_hbm, out_hbm,
          mmbuf, pv, xbuf, buf0, buf1, idx0, idx1, shared, sem0, sem1):
    cid = lax.axis_index("c")
    sid = lax.axis_index("s")
    w = sid * _NC + cid
    n = x_hbm.shape[0]

    mm_rows = n // _NS
    vmin = None
    vmax = None
    for sub in range(mm_rows // _MMSUB):
        pltpu.sync_copy(
            x_hbm.at[pl.ds(sid * mm_rows + sub * _MMSUB, _MMSUB)], mmbuf)

        def step(i, carry):
            cmin, cmax = carry
            xv = mmbuf[pl.ds(i * _L, _L)]
            return jnp.minimum(cmin, xv), jnp.maximum(cmax, xv)

        init = (mmbuf[pl.ds(0, _L)], mmbuf[pl.ds(0, _L)])
        smin, smax = lax.fori_loop(1, _MMSUB // _L, step, init, unroll=8)
        if vmin is None:
            vmin, vmax = smin, smax
        else:
            vmin = jnp.minimum(vmin, smin)
            vmax = jnp.maximum(vmax, smax)

    pv[0, 0, :] = vmin
    pv[0, 1, :] = vmax
    pltpu.sync_copy(pv.at[0], shared.at[sid])
    plsc.subcore_barrier()
    pltpu.sync_copy(shared, pv)

    vmin = pv[0, 0, :]
    vmax = pv[0, 1, :]
    for k in range(1, _NS):
        vmin = jnp.minimum(vmin, pv[k, 0, :])
        vmax = jnp.maximum(vmax, pv[k, 1, :])
    iota = lax.iota(jnp.int32, _L)
    for k in (1, 2, 4, 8):
        perm = jnp.bitwise_xor(iota, k)
        vmin = jnp.minimum(vmin, vmin.at[perm].get(mode="promise_in_bounds"))
        vmax = jnp.maximum(vmax, vmax.at[perm].get(mode="promise_in_bounds"))
    lo = vmin
    delta = (vmax - vmin) / jnp.float32(_NUM_BINS - 1)
    inv = jnp.float32(1.0) / delta

    rows = n // _NW
    nchunks = rows // _CHUNK
    base = w * rows

    pltpu.sync_copy(zeros_hbm, buf0)
    pltpu.sync_copy(zeros_hbm, buf1)

    onev = jnp.full((_L,), 1.0, dtype=jnp.float32)
    zerov = jnp.zeros((_L,), dtype=jnp.float32)

    def process(c, bufr, idxr, sem):
        @pl.when(c >= 2)
        def _():
            pltpu.make_async_copy(
                bufr, out_hbm.at[pl.ds(0, _CHUNK), :], sem).wait()
            for r in range(_GROUPS):
                rowv = iota + (r * _L)
                idxv = idxr[pl.ds(r * _L, _L)]
                plsc.store_scatter(bufr, [rowv, idxv], zerov)

        pltpu.sync_copy(x_hbm.at[pl.ds(base + c * _CHUNK, _CHUNK)], xbuf)
        for r in range(_GROUPS):
            xv = xbuf[pl.ds(r * _L, _L)]
            t = (xv - lo) * inv
            idxv = jnp.minimum(t.astype(jnp.int32) + 1, _NUM_BINS)
            idxr[pl.ds(r * _L, _L)] = idxv
            rowv = iota + (r * _L)
            plsc.store_scatter(bufr, [rowv, idxv], onev)
        pltpu.make_async_copy(
            bufr,
            out_hbm.at[pl.ds(base + c * _CHUNK, _CHUNK), :],
            sem,
        ).start()

    def body(c, _):
        slot = lax.rem(c, 2)

        @pl.when(slot == 0)
        def _():
            process(c, buf0, idx0, sem0)

        @pl.when(slot == 1)
        def _():
            process(c, buf1, idx1, sem1)

        return 0

    lax.fori_loop(0, nchunks, body, 0)
    pltpu.make_async_copy(buf0, out_hbm.at[pl.ds(0, _CHUNK), :], sem0).wait()
    pltpu.make_async_copy(buf1, out_hbm.at[pl.ds(0, _CHUNK), :], sem1).wait()


def kernel(feature):
    n = feature.shape[0]
    mesh = plsc.VectorSubcoreMesh(core_axis_name="c", subcore_axis_name="s")
    cparams = pltpu.CompilerParams(needs_layout_passes=False,
                                   use_tc_tiling_on_sc=True)
    expand = pl.kernel(
        _body,
        mesh=mesh,
        compiler_params=cparams,
        out_type=jax.ShapeDtypeStruct((n, _NCAT), jnp.float32),
        scratch_types=[
            pltpu.VMEM((_MMSUB,), jnp.float32),
            pltpu.VMEM((_NS, 2, _L), jnp.float32),
            pltpu.VMEM((_CHUNK,), jnp.float32),
            pltpu.VMEM((_CHUNK, _NCAT), jnp.float32),
            pltpu.VMEM((_CHUNK, _NCAT), jnp.float32),
            pltpu.VMEM((_CHUNK,), jnp.int32),
            pltpu.VMEM((_CHUNK,), jnp.int32),
            pltpu.VMEM_SHARED((_NS, 2, _L), jnp.float32),
            pltpu.SemaphoreType.DMA,
            pltpu.SemaphoreType.DMA,
        ],
    )
    zeros = jnp.zeros((_CHUNK, _NCAT), jnp.float32)
    return expand(feature, zeros)
